# Initial kernel scaffold; baseline (speedup 1.0000x reference)
import functools

import jax
import jax.numpy as jnp
from jax import lax
from jax.experimental import pallas as pl
from jax.experimental.pallas import tpu as pltpu
from jax.experimental.pallas import tpu_sc as plsc

N = 10000
E = 320000
D = 128

NC = 2
NS = 16
NW = NC * NS
EPW = E // NW
CH = 80
NCHUNK = EPW // CH
RPT = N // NS
DEGW = 16

_HI = jax.lax.Precision.HIGHEST

_sc_mesh = plsc.VectorSubcoreMesh(core_axis_name="c", subcore_axis_name="s")


@functools.partial(
    pl.kernel,
    out_type=jax.ShapeDtypeStruct((NC * N, DEGW), jnp.float32),
    mesh=_sc_mesh,
    scratch_types=[
        pltpu.VMEM((CH,), jnp.int32),
        pltpu.VMEM((CH, DEGW), jnp.float32),
        pltpu.VMEM_SHARED((N, DEGW), jnp.float32),
    ],
)
def _deg_kernel(dst_hbm, ones_hbm, zeros_hbm, out_hbm, dst_v, ones_v, deg_sh):
    cid = lax.axis_index("c")
    sid = lax.axis_index("s")
    wid = sid * NC + cid
    pltpu.sync_copy(zeros_hbm.at[pl.ds(sid * RPT, RPT)],
                    deg_sh.at[pl.ds(sid * RPT, RPT)])
    pltpu.sync_copy(ones_hbm, ones_v)
    plsc.subcore_barrier()
    base = wid * EPW

    def body(c, carry):
        off = pl.multiple_of(base + c * CH, 8)
        pltpu.sync_copy(dst_hbm.at[pl.ds(off, CH)], dst_v)
        pltpu.sync_copy(ones_v, deg_sh.at[dst_v], add=True)
        return carry

    lax.fori_loop(0, NCHUNK, body, 0)
    plsc.subcore_barrier()
    pltpu.sync_copy(deg_sh.at[pl.ds(sid * RPT, RPT)],
                    out_hbm.at[pl.ds(cid * N + sid * RPT, RPT)])


@functools.partial(
    pl.kernel,
    out_type=jax.ShapeDtypeStruct((NC * N, D), jnp.float32),
    mesh=_sc_mesh,
    scratch_types=[
        pltpu.VMEM((CH,), jnp.int32),
        pltpu.VMEM((CH,), jnp.int32),
        pltpu.VMEM((CH, D), jnp.float32),
        pltpu.VMEM_SHARED((N, D), jnp.float32),
        pltpu.SemaphoreType.DMA,
    ],
)
def _agg_kernel(xws_hbm, src_hbm, dst_hbm, zeros_hbm, out_hbm,
                src_v, dst_v, rows_v, acc_sh, sem):
    cid = lax.axis_index("c")
    sid = lax.axis_index("s")
    wid = sid * NC + cid
    pltpu.sync_copy(zeros_hbm.at[pl.ds(sid * RPT, RPT)],
                    acc_sh.at[pl.ds(sid * RPT, RPT)])
    plsc.subcore_barrier()
    base = wid * EPW

    def body(c, carry):
        off = pl.multiple_of(base + c * CH, 8)
        pltpu.sync_copy(src_hbm.at[pl.ds(off, CH)], src_v)
        pltpu.sync_copy(dst_hbm.at[pl.ds(off, CH)], dst_v)
        pltpu.async_copy(xws_hbm.at[src_v], rows_v, sem).wait()
        pltpu.sync_copy(rows_v, acc_sh.at[dst_v], add=True)
        return carry

    lax.fori_loop(0, NCHUNK, body, 0)
    plsc.subcore_barrier()
    pltpu.sync_copy(acc_sh.at[pl.ds(sid * RPT, RPT)],
                    out_hbm.at[pl.ds(cid * N + sid * RPT, RPT)])


def _prescale_body(nf_ref, wg_ref, wih_ref, whh_ref, bih_ref, bhh_ref,
                   degp_ref, xws_ref):
    W = wg_ref[...]
    gi = lax.dot_general(W, wih_ref[...], (((1,), (1,)), ((), ())),
                         precision=_HI) + bih_ref[...]
    gh = lax.dot_general(W, whh_ref[...], (((1,), (1,)), ((), ())),
                         precision=_HI) + bhh_ref[...]
    r = jax.nn.sigmoid(gi[:, :D] + gh[:, :D])
    z = jax.nn.sigmoid(gi[:, D:2 * D] + gh[:, D:2 * D])
    n = jnp.tanh(gi[:, 2 * D:] + r * gh[:, 2 * D:])
    w_ev = (1.0 - z) * n + z * W
    xw = jnp.dot(nf_ref[...], w_ev, precision=_HI)
    deg = degp_ref[0, :, :1] + degp_ref[1, :, :1] + 1.0
    xws_ref[...] = xw * lax.rsqrt(deg)


_prescale_call = pl.pallas_call(
    _prescale_body,
    out_shape=jax.ShapeDtypeStruct((N, D), jnp.float32),
)


def _out_body(acc_ref, xws_ref, degp_ref, wlin_ref, blin_ref, out_ref):
    deg = degp_ref[0, :, :1] + degp_ref[1, :, :1] + 1.0
    h = (acc_ref[0] + acc_ref[1] + xws_ref[...]) * lax.rsqrt(deg)
    zr = jnp.maximum(h, 0.0)
    out_ref[...] = lax.dot_general(zr, wlin_ref[...], (((1,), (1,)), ((), ())),
                                   precision=_HI) + blin_ref[...]


_out_call = pl.pallas_call(
    _out_body,
    out_shape=jax.ShapeDtypeStruct((N, D), jnp.float32),
)


def kernel(node_feat, src, dst, W_gcn, W_ih, W_hh, b_ih, b_hh, W_lin, b_lin):
    src = src.astype(jnp.int32)
    dst = dst.astype(jnp.int32)
    ones_chunk = jnp.ones((CH, DEGW), jnp.float32)
    zeros_deg = jnp.zeros((N, DEGW), jnp.float32)
    zeros_acc = jnp.zeros((N, D), jnp.float32)

    degp = _deg_kernel(dst, ones_chunk, zeros_deg).reshape(NC, N, DEGW)
    xws = _prescale_call(node_feat.astype(jnp.float32), W_gcn, W_ih, W_hh,
                         b_ih.reshape(1, 3 * D), b_hh.reshape(1, 3 * D), degp)
    acc = _agg_kernel(xws, src, dst, zeros_acc).reshape(NC, N, D)
    return _out_call(acc, xws, degp, W_lin, b_lin.reshape(1, D))


# speedup vs baseline: 20.3377x; 20.3377x over previous
import functools

import jax
import jax.numpy as jnp
from jax import lax
from jax.experimental import pallas as pl
from jax.experimental.pallas import tpu as pltpu
from jax.experimental.pallas import tpu_sc as plsc

N = 10000
E = 320000
D = 128

NC = 2
NS = 16
NW = NC * NS
EPW = E // NW
CH = 80
NCHUNK = EPW // CH
NP = 10240
RPT = NP // NS
DEGW = 16

_PREC = jax.lax.Precision.DEFAULT


def _deg_body(dst_hbm, out_hbm, dst_v, deg_v, acc_v, tmp_v, deg_sh):
    cid = lax.axis_index("c")
    sid = lax.axis_index("s")
    wid = sid * NC + cid
    zeros16 = jnp.zeros((16,), jnp.float32)
    ones16 = jnp.ones((16,), jnp.float32)

    def zbody(i, c):
        deg_v[pl.ds(pl.multiple_of(i * 16, 16), 16)] = zeros16
        return c
    lax.fori_loop(0, NP // 16, zbody, 0)

    pltpu.sync_copy(dst_hbm.at[pl.ds(wid * EPW, EPW)], dst_v)

    def ebody(i, c):
        iv = dst_v[pl.ds(pl.multiple_of(i * 16, 16), 16)]
        plsc.addupdate_scatter(deg_v, [iv], ones16)
        return c
    lax.fori_loop(0, EPW // 16, ebody, 0)

    pltpu.sync_copy(deg_v, deg_sh.at[sid])
    plsc.subcore_barrier()

    pltpu.sync_copy(deg_sh.at[0, pl.ds(sid * RPT, RPT)], acc_v)

    def mbody(j, c):
        pltpu.sync_copy(deg_sh.at[j, pl.ds(sid * RPT, RPT)], tmp_v)

        def abody(i, c2):
            sl = pl.ds(pl.multiple_of(i * 16, 16), 16)
            acc_v[sl] = acc_v[sl] + tmp_v[sl]
            return c2
        lax.fori_loop(0, RPT // 16, abody, 0)
        return c
    lax.fori_loop(1, NS, mbody, 0)
    pltpu.sync_copy(acc_v, out_hbm.at[pl.ds(cid * NP + sid * RPT, RPT)])


def _agg_body(xws_hbm, src_hbm, dst_hbm, zeros_hbm, out_hbm,
              src_v, dst_v, rows_v, acc_sh, sem):
    cid = lax.axis_index("c")
    sid = lax.axis_index("s")
    wid = sid * NC + cid
    pltpu.sync_copy(zeros_hbm.at[pl.ds(sid * RPT, RPT)],
                    acc_sh.at[pl.ds(sid * RPT, RPT)])
    plsc.subcore_barrier()
    base = wid * EPW

    def body(c, carry):
        off = pl.multiple_of(base + c * CH, 8)
        pltpu.sync_copy(src_hbm.at[pl.ds(off, CH)], src_v)
        pltpu.sync_copy(dst_hbm.at[pl.ds(off, CH)], dst_v)
        pltpu.async_copy(xws_hbm.at[src_v], rows_v, sem).wait()
        pltpu.sync_copy(rows_v, acc_sh.at[dst_v], add=True)
        return carry

    lax.fori_loop(0, NCHUNK, body, 0)
    plsc.subcore_barrier()
    pltpu.sync_copy(acc_sh.at[pl.ds(sid * RPT, RPT)],
                    out_hbm.at[pl.ds(cid * NP + sid * RPT, RPT)])


def _prescale_body(nf_ref, wg_ref, wih_ref, whh_ref, bih_ref, bhh_ref,
                   degp_ref, xws_ref):
    W = wg_ref[...]
    gi = lax.dot_general(W, wih_ref[...], (((1,), (1,)), ((), ())),
                         precision=_PREC) + bih_ref[...]
    gh = lax.dot_general(W, whh_ref[...], (((1,), (1,)), ((), ())),
                         precision=_PREC) + bhh_ref[...]
    r = jax.nn.sigmoid(gi[:, :D] + gh[:, :D])
    z = jax.nn.sigmoid(gi[:, D:2 * D] + gh[:, D:2 * D])
    n = jnp.tanh(gi[:, 2 * D:] + r * gh[:, 2 * D:])
    w_ev = (1.0 - z) * n + z * W
    xw = jnp.dot(nf_ref[...], w_ev, precision=_PREC)
    deg = degp_ref[0] + degp_ref[1] + 1.0
    xws_ref[...] = xw * lax.rsqrt(deg)


_prescale_call = pl.pallas_call(
    _prescale_body,
    out_shape=jax.ShapeDtypeStruct((NP, D), jnp.float32),
)


def _out_body(acc_ref, xws_ref, degp_ref, wlin_ref, blin_ref, out_ref):
    deg = degp_ref[0] + degp_ref[1] + 1.0
    h = (acc_ref[0] + acc_ref[1] + xws_ref[...]) * lax.rsqrt(deg)
    zr = jnp.maximum(h, 0.0)
    out_ref[...] = lax.dot_general(zr, wlin_ref[...], (((1,), (1,)), ((), ())),
                                   precision=_PREC) + blin_ref[...]


_out_call = pl.pallas_call(
    _out_body,
    out_shape=jax.ShapeDtypeStruct((NP, D), jnp.float32),
)


@functools.lru_cache(maxsize=1)
def _sc_kernels():
    mesh = plsc.VectorSubcoreMesh(core_axis_name="c", subcore_axis_name="s")
    deg_kernel = pl.kernel(
        _deg_body,
        out_type=jax.ShapeDtypeStruct((NC * NP,), jnp.float32),
        mesh=mesh,
        compiler_params=pltpu.CompilerParams(needs_layout_passes=False),
        scratch_types=[
            pltpu.VMEM((EPW,), jnp.int32),
            pltpu.VMEM((NP,), jnp.float32),
            pltpu.VMEM((RPT,), jnp.float32),
            pltpu.VMEM((RPT,), jnp.float32),
            pltpu.VMEM_SHARED((NS, NP), jnp.float32),
        ],
    )
    agg_kernel = pl.kernel(
        _agg_body,
        out_type=jax.ShapeDtypeStruct((NC * NP, D), jnp.float32),
        mesh=mesh,
        scratch_types=[
            pltpu.VMEM((CH,), jnp.int32),
            pltpu.VMEM((CH,), jnp.int32),
            pltpu.VMEM((CH, D), jnp.float32),
            pltpu.VMEM_SHARED((NP, D), jnp.float32),
            pltpu.SemaphoreType.DMA,
        ],
    )
    return deg_kernel, agg_kernel


def kernel(node_feat, src, dst, W_gcn, W_ih, W_hh, b_ih, b_hh, W_lin, b_lin):
    src = src.astype(jnp.int32)
    dst = dst.astype(jnp.int32)
    zeros_acc = jnp.zeros((NP, D), jnp.float32)
    nf = jnp.pad(node_feat.astype(jnp.float32), ((0, NP - N), (0, 0)))

    _deg_kernel, _agg_kernel = _sc_kernels()
    degp = _deg_kernel(dst).reshape(NC, NP, 1)
    xws = _prescale_call(nf, W_gcn, W_ih, W_hh,
                         b_ih.reshape(1, 3 * D), b_hh.reshape(1, 3 * D), degp)
    acc = _agg_kernel(xws, src, dst, zeros_acc).reshape(NC, NP, D)
    return _out_call(acc, xws, degp, W_lin, b_lin.reshape(1, D))[:N]
